# baseline (speedup 1.0000x reference)
import jax
import jax.numpy as jnp
from jax import lax
from jax.experimental import pallas as pl
from jax.experimental.pallas import tpu as pltpu
from jax.experimental.pallas import tpu_sc as plsc

B = 16384
N_ROWS = 1000000
HALF_ROWS = N_ROWS // 2
D_MODEL = 64
PAIR = 2 * D_MODEL
HALF_PAIRS = HALF_ROWS // 2
STRIDE1 = 1000.0

NC = 2
NS = 16
L = 16
NW = NC * NS
B_PER_W = B // NW
CHUNK = 64
N_CHUNKS = B_PER_W // CHUNK
GROUPS = CHUNK // L
D_REGS = D_MODEL // L


def _sc_body(c0_hbm, c1_hbm, lo_hbm, hi_hbm, out_hbm,
             c0_v, c1_v, jql_v, jqh_v, cq_v, pairs_v, rows_v, sem):
    wid = lax.axis_index("s") * NC + lax.axis_index("c")
    base = wid * B_PER_W

    pltpu.sync_copy(c0_hbm.at[pl.ds(base, B_PER_W)], c0_v)
    pltpu.sync_copy(c1_hbm.at[pl.ds(base, B_PER_W)], c1_v)

    for c in range(N_CHUNKS):
        for g in range(GROUPS):
            off = c * CHUNK + g * L
            v0 = c0_v[pl.ds(off, L)]
            v1 = c1_v[pl.ds(off, L)]
            idx = (v0 + STRIDE1 * v1).astype(jnp.int32)
            j = lax.shift_right_logical(idx, 1)
            jql_v[c, pl.ds(g * L, L)] = jnp.minimum(j, HALF_PAIRS - 1)
            jqh_v[c, pl.ds(g * L, L)] = jnp.maximum(j - HALF_PAIRS, 0)
            hsel = 1 - lax.shift_right_logical(idx - HALF_ROWS, 31)
            cq_v[pl.ds(off, L)] = (
                lax.shift_left(hsel, 15)
                + lax.bitwise_and(idx, 1) * D_MODEL
            )
        lo_cp = pltpu.async_copy(
            lo_hbm.at[jql_v.at[c]], pairs_v.at[0], sem
        )
        hi_cp = pltpu.async_copy(
            hi_hbm.at[jqh_v.at[c]], pairs_v.at[1], sem
        )
        lo_cp.wait()
        hi_cp.wait()

        def extract(i, _, c=c):
            code = cq_v[pl.ds(c * CHUNK + i, L)][0]
            hsel = lax.shift_right_logical(code, 15)
            h = lax.bitwise_and(code, 0x7FFF)
            for k in range(D_REGS):
                rows_v[c * CHUNK + i, pl.ds(k * L, L)] = (
                    pairs_v[hsel, i, pl.ds(h + k * L, L)]
                )
            return 0

        lax.fori_loop(0, CHUNK, extract, 0)

    pltpu.sync_copy(rows_v, out_hbm.at[pl.ds(base, B_PER_W)])


@jax.jit
def kernel(x_entity0, embeddings):
    mesh = plsc.VectorSubcoreMesh(core_axis_name="c", subcore_axis_name="s")
    run = pl.kernel(
        _sc_body,
        out_type=jax.ShapeDtypeStruct((B, D_MODEL), jnp.float32),
        mesh=mesh,
        scratch_types=[
            pltpu.VMEM((B_PER_W,), jnp.float32),
            pltpu.VMEM((B_PER_W,), jnp.float32),
            pltpu.VMEM((N_CHUNKS, CHUNK), jnp.int32),
            pltpu.VMEM((N_CHUNKS, CHUNK), jnp.int32),
            pltpu.VMEM((B_PER_W + L,), jnp.int32),
            pltpu.VMEM((2, CHUNK, PAIR), jnp.float32),
            pltpu.VMEM((B_PER_W, D_MODEL), jnp.float32),
            pltpu.SemaphoreType.DMA,
        ],
    )
    lo = embeddings[:HALF_ROWS].reshape(HALF_PAIRS, PAIR)
    hi = embeddings[HALF_ROWS:].reshape(HALF_PAIRS, PAIR)
    return run(x_entity0[:, 0], x_entity0[:, 1], lo, hi)


# speedup vs baseline: 3.3401x; 3.3401x over previous
import jax
import jax.numpy as jnp
from jax import lax
from jax.experimental import pallas as pl
from jax.experimental.pallas import tpu as pltpu
from jax.experimental.pallas import tpu_sc as plsc

B = 16384
D_MODEL = 64
STRIDE1 = 1000.0

NC = 2
NS = 16
L = 16
NW = NC * NS
B_PER_W = B // NW
GROUPS = B_PER_W // L
CHUNK = 64
N_CHUNKS = B_PER_W // CHUNK


def _sc_body(c0_hbm, c1_hbm, table_hbm, out_hbm,
             c0_v, c1_v, iq_v, rows_v, sems):
    wid = lax.axis_index("s") * NC + lax.axis_index("c")
    base = wid * B_PER_W

    pltpu.sync_copy(c0_hbm.at[pl.ds(base, B_PER_W)], c0_v)
    pltpu.sync_copy(c1_hbm.at[pl.ds(base, B_PER_W)], c1_v)

    for g in range(GROUPS):
        v0 = c0_v[pl.ds(g * L, L)]
        v1 = c1_v[pl.ds(g * L, L)]
        iq_v[pl.ds(g * L, L)] = (v0 + STRIDE1 * v1).astype(jnp.int32)

    def fire(g, _):
        vec = iq_v[pl.ds(g * L, L)]
        for j in range(L):
            pltpu.async_copy(
                table_hbm.at[vec[j]], rows_v.at[g * L + j], sems.at[j % 4]
            )
        return 0

    lax.fori_loop(0, GROUPS, fire, 0)
    for k in range(4):
        pltpu.make_async_copy(
            out_hbm.at[pl.ds(base + k * (B_PER_W // 4), B_PER_W // 4)],
            rows_v.at[pl.ds(k * (B_PER_W // 4), B_PER_W // 4)],
            sems.at[k],
        ).wait()

    pltpu.sync_copy(rows_v, out_hbm.at[pl.ds(base, B_PER_W)])


@jax.jit
def kernel(x_entity0, embeddings):
    mesh = plsc.VectorSubcoreMesh(core_axis_name="c", subcore_axis_name="s")
    run = pl.kernel(
        _sc_body,
        out_type=jax.ShapeDtypeStruct((B, D_MODEL), jnp.float32),
        mesh=mesh,
        scratch_types=[
            pltpu.VMEM((B_PER_W,), jnp.float32),
            pltpu.VMEM((B_PER_W,), jnp.float32),
            pltpu.VMEM((B_PER_W + L,), jnp.int32),
            pltpu.VMEM((B_PER_W, D_MODEL), jnp.float32),
            pltpu.SemaphoreType.DMA((4,)),
        ],
    )
    return run(x_entity0[:, 0], x_entity0[:, 1], embeddings)
